# baseline (speedup 1.0000x reference)
import functools

import jax
import jax.numpy as jnp
from jax import lax
from jax.experimental import pallas as pl
from jax.experimental.pallas import tpu as pltpu
from jax.experimental.pallas import tpu_sc as plsc


def _sc_broadcast(pos_table, batch, seq, d):
    info = plsc.get_sparse_core_info()
    nw = info.num_cores * info.num_subcores
    rows = batch * seq
    chunk = rows // nw
    mesh = plsc.VectorSubcoreMesh(core_axis_name="c", subcore_axis_name="s")

    @functools.partial(
        pl.kernel,
        mesh=mesh,
        out_type=jax.ShapeDtypeStruct((rows, d), pos_table.dtype),
    )
    def run(table_hbm, out_hbm):
        wid = lax.axis_index("s") * info.num_cores + lax.axis_index("c")
        dst = wid * chunk
        src = lax.rem(dst, seq)
        pltpu.sync_copy(
            table_hbm.at[pl.ds(src, chunk)],
            out_hbm.at[pl.ds(dst, chunk)],
        )

    return run(pos_table).reshape(batch, seq, d)


def kernel(x, pos_table):
    batch, seq, _ = x.shape
    d = pos_table.shape[1]
    return _sc_broadcast(pos_table, batch, seq, d)


# speedup vs baseline: 54.1163x; 54.1163x over previous
import functools

import jax
import jax.numpy as jnp
from jax import lax
from jax.experimental import pallas as pl
from jax.experimental.pallas import tpu as pltpu
from jax.experimental.pallas import tpu_sc as plsc

_TILE = 32
_NBUF = 2


def _sc_broadcast(pos_table, batch, seq, d):
    info = plsc.get_sparse_core_info()
    nw = info.num_cores * info.num_subcores
    ch_src = seq // nw
    nt = ch_src // _TILE
    mesh = plsc.VectorSubcoreMesh(core_axis_name="c", subcore_axis_name="s")

    @functools.partial(
        pl.kernel,
        mesh=mesh,
        out_type=jax.ShapeDtypeStruct((batch * seq, d), pos_table.dtype),
        scratch_types=[
            pltpu.VMEM((_NBUF, _TILE, d), pos_table.dtype),
            pltpu.SemaphoreType.DMA((_NBUF,)),
            pltpu.SemaphoreType.DMA((_NBUF,)),
        ],
    )
    def run(table_hbm, out_hbm, buf, in_sem, out_sem):
        wid = lax.axis_index("s") * info.num_cores + lax.axis_index("c")
        src0 = wid * ch_src

        def in_copy(t, slot):
            return pltpu.make_async_copy(
                table_hbm.at[pl.ds(src0 + t * _TILE, _TILE)],
                buf.at[slot],
                in_sem.at[slot],
            )

        def out_copy(t, slot, b):
            return pltpu.make_async_copy(
                buf.at[slot],
                out_hbm.at[pl.ds(b * seq + src0 + t * _TILE, _TILE)],
                out_sem.at[slot],
            )

        in_copy(0, 0).start()
        for t in range(nt):
            slot = t % _NBUF
            in_copy(t, slot).wait()
            for b in range(batch):
                out_copy(t, slot, b).start()
            if t + 1 < nt:
                nslot = (t + 1) % _NBUF
                if t >= 1:
                    for b in range(batch):
                        out_copy(t - 1, nslot, b).wait()
                in_copy(t + 1, nslot).start()
        for b in range(batch):
            out_copy(nt - 1, (nt - 1) % _NBUF, b).wait()

    return run(pos_table).reshape(batch, seq, d)


def kernel(x, pos_table):
    batch, seq, _ = x.shape
    d = pos_table.shape[1]
    return _sc_broadcast(pos_table, batch, seq, d)
